# baseline (speedup 1.0000x reference)
import functools

import jax
import jax.numpy as jnp
from jax import lax
from jax.experimental import pallas as pl
from jax.experimental.pallas import tpu as pltpu
from jax.experimental.pallas import tpu_sc as plsc

BATCH = 16384
FIELDS = 100
TOTAL = BATCH * FIELDS
VOCAB = 1_000_000

_info = plsc.get_sparse_core_info()
NC, NS = _info.num_cores, _info.num_subcores
NW = NC * NS
B_PER_W = TOTAL // NW
CH = 3_200
N_CHUNKS = B_PER_W // CH
MAIN = 999_424
TAIL = VOCAB - MAIN
STAGE_CH = 15_616
N_STAGE = MAIN // STAGE_CH


def _gather_kernel(table_hbm, tail_hbm, idx_hbm, out_hbm, tab_sp, idx_v0,
                   idx_v1, rows_v0, rows_v1, stage_v0, stage_v1, sem_i0,
                   sem_i1, sem_g0, sem_g1, sem_o0, sem_o1):
    idx_v = (idx_v0, idx_v1)
    rows_v = (rows_v0, rows_v1)
    sem_i = (sem_i0, sem_i1)
    sem_g = (sem_g0, sem_g1)
    sem_o = (sem_o0, sem_o1)
    sid = lax.axis_index("s")
    wid = sid * NC + lax.axis_index("c")
    base = wid * B_PER_W

    def idx_copy(c):
        s = c % 2
        return pltpu.make_async_copy(
            idx_hbm.at[pl.ds(base + c * CH, CH)], idx_v[s], sem_i[s])

    def gather_copy(c):
        s = c % 2
        return pltpu.make_async_copy(
            tab_sp.at[idx_v[s]], rows_v[s], sem_g[s])

    def out_copy(c):
        s = c % 2
        return pltpu.make_async_copy(
            rows_v[s], out_hbm.at[pl.ds(base + c * CH, CH)], sem_o[s])

    idx_copy(0).start()
    idx_copy(1).start()

    K_STAGE = N_STAGE // NS
    stage_b = (stage_v0, stage_v1)
    sem_h = (sem_g0, sem_g1)
    sem_t = (sem_o0, sem_o1)

    def h2t(k):
        s = k % 2
        off = (sid + NS * k) * STAGE_CH
        return pltpu.make_async_copy(
            table_hbm.at[pl.ds(off, STAGE_CH)], stage_b[s], sem_h[s])

    def t2s(k):
        s = k % 2
        off = (sid + NS * k) * STAGE_CH
        return pltpu.make_async_copy(
            stage_b[s], tab_sp.at[pl.ds(off, STAGE_CH)], sem_t[s])

    h2t(0).start()
    for k in range(K_STAGE):
        h2t(k).wait()
        t2s(k).start()
        if k + 1 < K_STAGE:
            if k >= 1:
                t2s(k - 1).wait()
            h2t(k + 1).start()
    t2s(K_STAGE - 2).wait()
    t2s(K_STAGE - 1).wait()

    @pl.when(sid == 0)
    def _stage_tail():
        pltpu.sync_copy(tail_hbm, stage_v0.at[pl.ds(0, TAIL)])
        pltpu.sync_copy(stage_v0.at[pl.ds(0, TAIL)], tab_sp.at[pl.ds(MAIN, TAIL)])

    plsc.subcore_barrier()

    for c in range(N_CHUNKS):
        if 2 <= c + 1 < N_CHUNKS:
            idx_copy(c + 1).start()
        if c >= 2:
            out_copy(c - 2).wait()
        idx_copy(c).wait()
        gather_copy(c).start()
        gather_copy(c).wait()
        out_copy(c).start()
    out_copy(N_CHUNKS - 2).wait()
    out_copy(N_CHUNKS - 1).wait()


@jax.jit
def kernel(inputs, table):
    idx = inputs.T.reshape(-1)
    main1d = lax.slice(table, (0, 0), (MAIN, 1)).reshape(-1)
    tail1d = lax.slice(table, (MAIN, 0), (VOCAB, 1)).reshape(-1)
    mesh = plsc.VectorSubcoreMesh(core_axis_name="c", subcore_axis_name="s")
    out = pl.kernel(
        _gather_kernel,
        mesh=mesh,
        out_type=jax.ShapeDtypeStruct((TOTAL,), jnp.float32),
        scratch_types=[
            pltpu.VMEM_SHARED((VOCAB,), jnp.float32),
            pltpu.VMEM((CH,), jnp.int32),
            pltpu.VMEM((CH,), jnp.int32),
            pltpu.VMEM((CH,), jnp.float32),
            pltpu.VMEM((CH,), jnp.float32),
            pltpu.VMEM((STAGE_CH,), jnp.float32),
            pltpu.VMEM((STAGE_CH,), jnp.float32),
            pltpu.SemaphoreType.DMA,
            pltpu.SemaphoreType.DMA,
            pltpu.SemaphoreType.DMA,
            pltpu.SemaphoreType.DMA,
            pltpu.SemaphoreType.DMA,
            pltpu.SemaphoreType.DMA,
        ],
    )(main1d, tail1d, idx)
    return out.reshape(FIELDS, BATCH).T


# speedup vs baseline: 1.0178x; 1.0178x over previous
import jax
import jax.numpy as jnp
from jax import lax
from jax.experimental import pallas as pl
from jax.experimental.pallas import tpu as pltpu
from jax.experimental.pallas import tpu_sc as plsc

BATCH = 16384
FIELDS = 100
TOTAL = BATCH * FIELDS
VOCAB = 1_000_000

_info = plsc.get_sparse_core_info()
NC, NS = _info.num_cores, _info.num_subcores
NW = NC * NS
B_PER_W = TOTAL // NW
CH = 6_400
N_CHUNKS = B_PER_W // CH
MAIN = 999_424
TAIL = VOCAB - MAIN
STAGE_CH = 15_616
N_STAGE = MAIN // STAGE_CH


def _gather_kernel(table_hbm, tail_hbm, idx_hbm, out_hbm, tab_sp, idx_v0,
                   idx_v1, rows_v0, rows_v1, stage_v0, stage_v1, sem_i0,
                   sem_i1, sem_g0, sem_g1, sem_o0, sem_o1):
    idx_v = (idx_v0, idx_v1)
    rows_v = (rows_v0, rows_v1)
    sem_i = (sem_i0, sem_i1)
    sem_g = (sem_g0, sem_g1)
    sem_o = (sem_o0, sem_o1)
    sid = lax.axis_index("s")
    wid = sid * NC + lax.axis_index("c")
    base = wid * B_PER_W

    def idx_copy(c):
        s = c % 2
        return pltpu.make_async_copy(
            idx_hbm.at[pl.ds(base + c * CH, CH)], idx_v[s], sem_i[s])

    def gather_copy(c):
        s = c % 2
        return pltpu.make_async_copy(
            tab_sp.at[idx_v[s]], rows_v[s], sem_g[s])

    def out_copy(c):
        s = c % 2
        return pltpu.make_async_copy(
            rows_v[s], out_hbm.at[pl.ds(base + c * CH, CH)], sem_o[s])

    idx_copy(0).start()
    idx_copy(1).start()

    K_STAGE = N_STAGE // NS
    stage_b = (stage_v0, stage_v1)
    sem_h = (sem_g0, sem_g1)
    sem_t = (sem_o0, sem_o1)

    def h2t(k):
        s = k % 2
        off = (sid + NS * k) * STAGE_CH
        return pltpu.make_async_copy(
            table_hbm.at[pl.ds(off, STAGE_CH)], stage_b[s], sem_h[s])

    def t2s(k):
        s = k % 2
        off = (sid + NS * k) * STAGE_CH
        return pltpu.make_async_copy(
            stage_b[s], tab_sp.at[pl.ds(off, STAGE_CH)], sem_t[s])

    h2t(0).start()
    for k in range(K_STAGE):
        h2t(k).wait()
        t2s(k).start()
        if k + 1 < K_STAGE:
            if k >= 1:
                t2s(k - 1).wait()
            h2t(k + 1).start()
    t2s(K_STAGE - 2).wait()
    t2s(K_STAGE - 1).wait()

    @pl.when(sid == 0)
    def _stage_tail():
        pltpu.sync_copy(tail_hbm, stage_v0.at[pl.ds(0, TAIL)])
        pltpu.sync_copy(stage_v0.at[pl.ds(0, TAIL)], tab_sp.at[pl.ds(MAIN, TAIL)])

    plsc.subcore_barrier()

    for c in range(N_CHUNKS):
        if 2 <= c + 1 < N_CHUNKS:
            idx_copy(c + 1).start()
        if c >= 2:
            out_copy(c - 2).wait()
        idx_copy(c).wait()
        gather_copy(c).start()
        gather_copy(c).wait()
        out_copy(c).start()
    out_copy(N_CHUNKS - 2).wait()
    out_copy(N_CHUNKS - 1).wait()


@jax.jit
def kernel(inputs, table):
    idx = inputs.T.reshape(-1)
    main1d = lax.slice(table, (0, 0), (MAIN, 1)).reshape(-1)
    tail1d = lax.slice(table, (MAIN, 0), (VOCAB, 1)).reshape(-1)
    mesh = plsc.VectorSubcoreMesh(core_axis_name="c", subcore_axis_name="s")
    out = pl.kernel(
        _gather_kernel,
        mesh=mesh,
        out_type=jax.ShapeDtypeStruct((TOTAL,), jnp.float32),
        scratch_types=[
            pltpu.VMEM_SHARED((VOCAB,), jnp.float32),
            pltpu.VMEM((CH,), jnp.int32),
            pltpu.VMEM((CH,), jnp.int32),
            pltpu.VMEM((CH,), jnp.float32),
            pltpu.VMEM((CH,), jnp.float32),
            pltpu.VMEM((STAGE_CH,), jnp.float32),
            pltpu.VMEM((STAGE_CH,), jnp.float32),
            pltpu.SemaphoreType.DMA,
            pltpu.SemaphoreType.DMA,
            pltpu.SemaphoreType.DMA,
            pltpu.SemaphoreType.DMA,
            pltpu.SemaphoreType.DMA,
            pltpu.SemaphoreType.DMA,
        ],
    )(main1d, tail1d, idx)
    return out.reshape(FIELDS, BATCH).T
